# baseline (speedup 1.0000x reference)
import functools

import jax
import jax.numpy as jnp
from jax import lax
from jax.experimental import pallas as pl
from jax.experimental.pallas import tpu as pltpu
from jax.experimental.pallas import tpu_sc as plsc

_L = 16


def _pick_chunk(rows_per_worker: int) -> int:
    for c in (512, 384, 256, 128):
        if rows_per_worker % c == 0:
            return c
    raise ValueError(f"rows_per_worker {rows_per_worker} not divisible by 128")


@functools.lru_cache(maxsize=None)
def _build_gather(n_rows: int, vocab: int, dim: int, hist: int):
    info = plsc.get_sparse_core_info()
    nc, ns = info.num_cores, info.num_subcores
    nw = nc * ns
    assert n_rows % nw == 0
    rpw = n_rows // nw
    chunk = _pick_chunk(rpw)
    nch = rpw // chunk
    ng = chunk // 128
    assert dim % _L == 0

    mesh = plsc.VectorSubcoreMesh(core_axis_name="c", subcore_axis_name="s")

    @functools.partial(
        pl.kernel,
        out_type=jax.ShapeDtypeStruct((n_rows, dim), jnp.float32),
        mesh=mesh,
        compiler_params=pltpu.CompilerParams(use_tc_tiling_on_sc=False),
        scratch_types=[
            pltpu.VMEM((chunk,), jnp.int32),
            pltpu.VMEM((chunk, dim), jnp.float32),
            pltpu.VMEM((_L,), jnp.float32),
            pltpu.SemaphoreType.DMA,
        ],
    )
    def body(idx_hbm, table_hbm, zs_hbm, out_hbm, idx_v, rows_v, zs_v, sem):
        wid = lax.axis_index("s") * nc + lax.axis_index("c")
        pltpu.sync_copy(zs_hbm, zs_v)

        def do_chunk(g, carry):
            base = pl.multiple_of(wid * rpw + g * chunk, 128)
            pltpu.sync_copy(idx_hbm.at[pl.ds(base, chunk)], idx_v)
            copies = [
                pltpu.async_copy(
                    table_hbm.at[idx_v.at[pl.ds(j * 128, 128)]],
                    rows_v.at[pl.ds(j * 128, 128)],
                    sem,
                )
                for j in range(ng)
            ]
            for c in copies:
                c.wait()

            @pl.when((wid == 0) & (g == 0))
            def _fix():
                zs = zs_v[...]

                def rowfix(i, c2):
                    for k in range(dim // _L):
                        sl = pl.ds(k * _L, _L)
                        rows_v[i, sl] = rows_v[i, sl] * zs
                    return c2

                lax.fori_loop(0, hist, rowfix, 0)

            pltpu.sync_copy(rows_v, out_hbm.at[pl.ds(base, chunk)])
            return carry

        lax.fori_loop(0, nch, do_chunk, 0)

    return body


def kernel(y, table, is_train):
    b, h = y.shape
    vocab, dim = table.shape
    flat = y.reshape(-1).astype(jnp.int32)
    shifted = jnp.concatenate([jnp.zeros((h,), jnp.int32), flat[:-h]])
    train = is_train != 0
    idx = jnp.where(train, shifted, flat)
    zscale = jnp.where(train, jnp.zeros((_L,), jnp.float32),
                       jnp.ones((_L,), jnp.float32))
    out_flat = _build_gather(b * h, vocab, dim, h)(idx, table, zscale)
    return out_flat.reshape(b, h, dim)


# speedup vs baseline: 1.0338x; 1.0338x over previous
import functools

import jax
import jax.numpy as jnp
from jax import lax
from jax.experimental import pallas as pl
from jax.experimental.pallas import tpu as pltpu
from jax.experimental.pallas import tpu_sc as plsc

_L = 16


def _pick_chunk(rows_per_worker: int) -> int:
    for c in (512, 384, 256, 128):
        if rows_per_worker % c == 0:
            return c
    raise ValueError(f"rows_per_worker {rows_per_worker} not divisible by 128")


@functools.lru_cache(maxsize=None)
def _build_gather(n_rows: int, vocab: int, dim: int, hist: int):
    info = plsc.get_sparse_core_info()
    nc, ns = info.num_cores, info.num_subcores
    nw = nc * ns
    assert n_rows % nw == 0
    rpw = n_rows // nw
    chunk = _pick_chunk(rpw)
    nch = rpw // chunk
    ng = chunk // 128
    assert dim % _L == 0

    mesh = plsc.VectorSubcoreMesh(core_axis_name="c", subcore_axis_name="s")

    assert nch % 2 == 0
    npair = nch // 2

    @functools.partial(
        pl.kernel,
        out_type=jax.ShapeDtypeStruct((n_rows, dim), jnp.float32),
        mesh=mesh,
        compiler_params=pltpu.CompilerParams(use_tc_tiling_on_sc=False),
        scratch_types=[
            pltpu.VMEM((chunk,), jnp.int32),
            pltpu.VMEM((chunk, dim), jnp.float32),
            pltpu.VMEM((chunk,), jnp.int32),
            pltpu.VMEM((chunk, dim), jnp.float32),
            pltpu.VMEM((_L,), jnp.float32),
            pltpu.SemaphoreType.DMA,
            pltpu.SemaphoreType.DMA,
        ],
    )
    def body(idx_hbm, table_hbm, zs_hbm, out_hbm,
             idx_a, rows_a, idx_b, rows_b, zs_v, sem_a, sem_b):
        wid = lax.axis_index("s") * nc + lax.axis_index("c")
        w0 = wid * rpw
        pltpu.sync_copy(zs_hbm, zs_v)

        def issue(idx_v, rows_v, sem, base):
            pltpu.sync_copy(idx_hbm.at[pl.ds(base, chunk)], idx_v)
            for k in range(ng):
                pltpu.async_copy(
                    table_hbm.at[idx_v.at[pl.ds(k * 128, 128)]],
                    rows_v.at[pl.ds(k * 128, 128)],
                    sem,
                )

        def drain(idx_v, rows_v, sem):
            for k in range(ng):
                pltpu.make_async_copy(
                    table_hbm.at[idx_v.at[pl.ds(k * 128, 128)]],
                    rows_v.at[pl.ds(k * 128, 128)],
                    sem,
                ).wait()

        issue(idx_a, rows_a, sem_a, pl.multiple_of(w0, 128))

        def pair(j, carry):
            e_base = pl.multiple_of(w0 + (2 * j) * chunk, 128)
            o_base = pl.multiple_of(w0 + (2 * j + 1) * chunk, 128)
            issue(idx_b, rows_b, sem_b, o_base)
            drain(idx_a, rows_a, sem_a)

            @pl.when((wid == 0) & (j == 0))
            def _fix():
                zs = zs_v[...]

                def rowfix(i, c2):
                    for k in range(dim // _L):
                        sl = pl.ds(k * _L, _L)
                        rows_a[i, sl] = rows_a[i, sl] * zs
                    return c2

                lax.fori_loop(0, hist, rowfix, 0)

            pltpu.sync_copy(rows_a, out_hbm.at[pl.ds(e_base, chunk)])

            @pl.when(j < npair - 1)
            def _next():
                issue(idx_a, rows_a, sem_a,
                      pl.multiple_of(w0 + (2 * j + 2) * chunk, 128))

            drain(idx_b, rows_b, sem_b)
            pltpu.sync_copy(rows_b, out_hbm.at[pl.ds(o_base, chunk)])
            return carry

        lax.fori_loop(0, npair, pair, 0)

    return body


def kernel(y, table, is_train):
    b, h = y.shape
    vocab, dim = table.shape
    flat = y.reshape(-1).astype(jnp.int32)
    shifted = jnp.concatenate([jnp.zeros((h,), jnp.int32), flat[:-h]])
    train = is_train != 0
    idx = jnp.where(train, shifted, flat)
    zscale = jnp.where(train, jnp.zeros((_L,), jnp.float32),
                       jnp.ones((_L,), jnp.float32))
    out_flat = _build_gather(b * h, vocab, dim, h)(idx, table, zscale)
    return out_flat.reshape(b, h, dim)
